# baseline (speedup 1.0000x reference)
import functools

import jax
import jax.numpy as jnp
from jax import lax
from jax.experimental import pallas as pl
from jax.experimental.pallas import tpu as pltpu
from jax.experimental.pallas import tpu_sc as plsc

EMBED_DIM = 32
CHUNK = 128
NBUF = 5


@functools.cache
def _build(B, V, D):
    info = plsc.get_sparse_core_info()
    num_workers = info.num_cores * info.num_subcores
    b_per_w = B // num_workers
    assert b_per_w % CHUNK == 0
    nsteps = b_per_w // CHUNK
    assert nsteps % NBUF == 0
    nouter = nsteps // NBUF

    mesh = plsc.VectorSubcoreMesh(core_axis_name="c", subcore_axis_name="s")

    @functools.partial(
        pl.kernel,
        mesh=mesh,
        compiler_params=pltpu.CompilerParams(use_tc_tiling_on_sc=False),
        out_type=jax.ShapeDtypeStruct((B, D), jnp.float32),
        scratch_types=(
            [pltpu.VMEM((b_per_w,), jnp.int32),
             pltpu.VMEM((NBUF, CHUNK, D), jnp.float32)]
            + [pltpu.SemaphoreType.DMA] * (2 * NBUF)
        ),
    )
    def gather_kernel(idx_hbm, table_hbm, out_hbm, idx_v, bufs, *sems):
        gsem = sems[:NBUF]
        osem = sems[NBUF:]
        wid = lax.axis_index("s") * info.num_cores + lax.axis_index("c")
        base = wid * b_per_w

        pltpu.sync_copy(idx_hbm.at[pl.ds(base, b_per_w)], idx_v)

        def outer(o, carry):
            for b in range(NBUF):
                @pl.when(o > 0)
                def _(b=b):
                    pltpu.make_async_copy(
                        bufs.at[b], out_hbm.at[pl.ds(base, CHUNK)], osem[b]
                    ).wait()
                c = o * NBUF + b
                pltpu.make_async_copy(
                    table_hbm.at[idx_v.at[pl.ds(c * CHUNK, CHUNK)]],
                    bufs.at[b],
                    gsem[b],
                ).start()
            for b in range(NBUF):
                c = o * NBUF + b
                pltpu.make_async_copy(
                    table_hbm.at[idx_v.at[pl.ds(c * CHUNK, CHUNK)]],
                    bufs.at[b],
                    gsem[b],
                ).wait()
                pltpu.make_async_copy(
                    bufs.at[b], out_hbm.at[pl.ds(base + c * CHUNK, CHUNK)], osem[b]
                ).start()
            return carry

        lax.fori_loop(0, nouter, outer, 0)
        for b in range(NBUF):
            pltpu.make_async_copy(
                bufs.at[b], out_hbm.at[pl.ds(base, CHUNK)], osem[b]
            ).wait()

    return gather_kernel


def kernel(x, table):
    B = x.shape[0] * x.shape[1]
    V, D = table.shape
    fn = _build(B, V, D)
    flat = x.reshape(B).astype(jnp.int32)
    out = fn(flat, table)
    return out.reshape(x.shape[0], x.shape[1], D)


# speedup vs baseline: 1.0103x; 1.0103x over previous
import functools

import jax
import jax.numpy as jnp
from jax import lax
from jax.experimental import pallas as pl
from jax.experimental.pallas import tpu as pltpu
from jax.experimental.pallas import tpu_sc as plsc

EMBED_DIM = 32
ROWS4 = 4
LANES = 16


@functools.cache
def _build(NI, NJ, V):
    info = plsc.get_sparse_core_info()
    NC = info.num_cores
    NW = NC * info.num_subcores
    IB = NI // NW
    assert IB == 128 and NJ % 2 == 0

    mesh = plsc.VectorSubcoreMesh(core_axis_name="c", subcore_axis_name="s")

    @functools.partial(
        pl.kernel,
        mesh=mesh,
        compiler_params=pltpu.CompilerParams(
            use_tc_tiling_on_sc=False, needs_layout_passes=False
        ),
        out_type=jax.ShapeDtypeStruct((NJ, EMBED_DIM, NI), jnp.float32),
        scratch_types=(
            [
                pltpu.VMEM((NJ, IB), jnp.int32),
                pltpu.VMEM((NJ, IB), jnp.int32),
                pltpu.VMEM((2, IB, 128), jnp.float32),
                pltpu.VMEM((2, EMBED_DIM, IB), jnp.float32),
            ]
            + [pltpu.SemaphoreType.DMA] * 4
        ),
    )
    def emb_kernel(xt_hbm, tab_hbm, out_hbm, gidx, offv, gbuf, obuf,
                   gsem0, gsem1, osem0, osem1):
        gsem = (gsem0, gsem1)
        osem = (osem0, osem1)
        wid = lax.axis_index("s") * NC + lax.axis_index("c")
        ibase = wid * IB

        pltpu.sync_copy(xt_hbm.at[:, pl.ds(ibase, IB)], gidx)

        def prep(t, carry):
            for k in range(8):
                s = pl.ds(k * LANES, LANES)
                v = gidx[t, s]
                offv[t, s] = (v & 3) * 32
                gidx[t, s] = v >> 2
            return carry

        lax.fori_loop(0, NJ, prep, 0)

        def gather_start(j, b):
            pltpu.make_async_copy(
                tab_hbm.at[gidx.at[j]], gbuf.at[b], gsem[b]
            ).start()

        def gather_wait(b):
            pltpu.make_async_copy(
                tab_hbm.at[gidx.at[0]], gbuf.at[b], gsem[b]
            ).wait()

        def out_start(j, b):
            pltpu.make_async_copy(
                obuf.at[b], out_hbm.at[j, :, pl.ds(ibase, IB)], osem[b]
            ).start()

        def out_wait(b):
            pltpu.make_async_copy(
                obuf.at[b], out_hbm.at[0, :, pl.ds(ibase, IB)], osem[b]
            ).wait()

        def extract(j, b):
            def col(c, carry):
                for k in range(IB // LANES):
                    rows = lax.iota(jnp.int32, LANES) + (k * LANES)
                    cols = offv[j, pl.ds(k * LANES, LANES)] + c
                    obuf[b, c, pl.ds(k * LANES, LANES)] = plsc.load_gather(
                        gbuf.at[b], [rows, cols]
                    )
                return carry

            lax.fori_loop(0, EMBED_DIM, col, 0)

        gather_start(0, 0)
        gather_start(1, 1)

        def step(o, carry):
            for b in range(2):
                j = o * 2 + b
                gather_wait(b)
                pl.when(j >= 2)(lambda b=b: out_wait(b))
                extract(j, b)
                out_start(j, b)
                pl.when(j + 2 < NJ)(lambda j=j, b=b: gather_start(j + 2, b))
            return carry

        lax.fori_loop(0, NJ // 2, step, 0)
        out_wait(0)
        out_wait(1)

    return emb_kernel


def kernel(x, table):
    NI, NJ = x.shape
    V, D = table.shape
    fn = _build(NI, NJ, V)
    xt = x.T.astype(jnp.int32)
    tab4 = table.reshape(V // ROWS4, D * ROWS4)
    out_t = fn(xt, tab4)
    return out_t.transpose(2, 0, 1)
